# baseline (speedup 1.0000x reference)
import functools

import jax
import jax.numpy as jnp
from jax.experimental import pallas as pl
from jax.experimental.pallas import tpu as pltpu

T, A, D, H, E = 8, 2, 1024, 2048, 8
HB = 1024
NH = H // HB


def _map_body(ei_ref, m_ref):
    vals = [ei_ref[j] for j in range(T * A)]
    used = [functools.reduce(jnp.logical_or, [v == e for v in vals])
            for e in range(E)]
    mn = jnp.int32(-1)
    for e in reversed(range(E)):
        mn = jnp.where(used[e], jnp.int32(e), mn)
    run = jnp.int32(-1)
    for e in range(E):
        run = jnp.where(used[e], jnp.int32(e), run)
        m_ref[e] = jnp.where(run >= 0, run, mn)


def _expert_map(ei_flat):
    return pl.pallas_call(
        _map_body,
        in_specs=[pl.BlockSpec(memory_space=pltpu.SMEM)],
        out_specs=pl.BlockSpec(memory_space=pltpu.SMEM),
        out_shape=jax.ShapeDtypeStruct((E,), jnp.int32),
    )(ei_flat)


def _ffn_body(m_ref, x_ref, ei_ref, wg_ref, wu_ref, wd_ref, out_ref):
    h = pl.program_id(0)
    e = pl.program_id(1)

    @pl.when((e == 0) & (h == 0))
    def _init():
        out_ref[...] = jnp.zeros_like(out_ref)

    mask = ei_ref[...] == e

    @pl.when(jnp.any(mask))
    def _compute():
        xb = x_ref[...]
        dn = (((1,), (1,)), ((), ()))
        g = jax.lax.dot_general(xb, wg_ref[0], dn,
                                preferred_element_type=jnp.float32)
        u = jax.lax.dot_general(xb, wu_ref[0], dn,
                                preferred_element_type=jnp.float32)
        act = (g * jax.lax.logistic(g)) * u
        y = jax.lax.dot_general(act, wd_ref[0], dn,
                                preferred_element_type=jnp.float32)
        out_ref[...] += jnp.where(mask, y, 0.0)


@jax.jit
def kernel(x, expert_indices, w_gate, w_up, w_down):
    x2 = jnp.repeat(x, A, axis=0)
    ei_flat = expert_indices.reshape(T * A).astype(jnp.int32)
    ei2 = ei_flat.reshape(T * A, 1)
    emap = _expert_map(ei_flat)

    grid = (NH, E)
    out = pl.pallas_call(
        _ffn_body,
        grid_spec=pltpu.PrefetchScalarGridSpec(
            num_scalar_prefetch=1,
            grid=grid,
            in_specs=[
                pl.BlockSpec((T * A, D), lambda h, e, m: (0, 0)),
                pl.BlockSpec((T * A, 1), lambda h, e, m: (0, 0)),
                pl.BlockSpec((1, HB, D), lambda h, e, m: (m[e], h, 0)),
                pl.BlockSpec((1, HB, D), lambda h, e, m: (m[e], h, 0)),
                pl.BlockSpec((1, D, HB), lambda h, e, m: (m[e], 0, h)),
            ],
            out_specs=pl.BlockSpec((T * A, D), lambda h, e, m: (0, 0)),
        ),
        out_shape=jax.ShapeDtypeStruct((T * A, D), jnp.float32),
    )(emap, x2, ei2, w_gate, w_up, w_down)
    return out.reshape(T, A, D)
